# baseline (speedup 1.0000x reference)
import functools

import jax
import jax.numpy as jnp
from jax import lax
from jax.experimental import pallas as pl
from jax.experimental.pallas import tpu as pltpu
from jax.experimental.pallas import tpu_sc as plsc

_B, _N, _K = 64, 196, 4096
_NB = 4
_S = 140

_NW = 32
_SC_BASE = _S * _B
_SC_TOK = (_N - _S) * _B
_RW = -(-_SC_TOK // _NW)
_RW = -(-_RW // 16) * 16
_TOK_PAD = _NW * _RW
_NCH = _RW // 16
_TROWS = _N * 8 * 32 * 8

_P0 = -2.133809518617725 - 127.0
_P1 = 3.0107182106126715
_P2 = -1.029486182176504
_P3 = 0.15391242162585833
_LN2 = 0.6931471805599453


def _tc_body(mask_ref, ps_ref, pt_ref, sum_ref, cnt_ref):
    i = pl.program_id(0)

    @pl.when(i == 0)
    def _():
        sum_ref[0, 0] = 0.0
        cnt_ref[0, 0] = 0.0

    m = mask_ref[:, :, 0:1] > 0.0
    safe = jnp.where(m, ps_ref[...], 1.0)
    sum_ref[0, 0] += jnp.sum(pt_ref[...] * jnp.log(safe))
    cnt_ref[0, 0] += jnp.sum(mask_ref[:, :, 0])


def _sc_body(mask_hbm, ps_hbm, pt_hbm, out_hbm,
             mask_v, tok_v, idx0, idx1, ps0, ps1, pt0, pt1, accv, csbuf,
             sps0, sps1, spt0, spt1):
    w = lax.axis_index("s") * 2 + lax.axis_index("c")
    base_tok = w * _RW

    pltpu.sync_copy(mask_hbm.at[pl.ds(base_tok, _RW)], mask_v)

    zeros16 = jnp.zeros((16,), jnp.int32)
    for j in range(8):
        idx0[pl.ds(j * 16, 16)] = zeros16
        idx1[pl.ds(j * 16, 16)] = zeros16
    for j in range(_NCH):
        tok_v[j] = zeros16

    iota16 = lax.iota(jnp.int32, 16)

    def _prefix16(v):
        cur = v
        for d in (1, 2, 4, 8):
            csbuf[...] = cur
            sh = plsc.load_gather(csbuf, [jnp.maximum(iota16 - d, 0)])
            cur = cur + jnp.where(iota16 >= d, sh, 0)
        return cur

    def _compact(j, cum):
        v = mask_v[pl.ds(j * 16, 16)]
        ids = _SC_BASE + base_tok + j * 16 + iota16
        cs = _prefix16(v)
        pos = cum + cs - v
        plsc.store_scatter(tok_v, [pos >> 4, pos & 15], ids, mask=v > 0)
        return cum + lax.squeeze(lax.slice(cs, (15,), (16,)), (0,))

    c_w = jnp.int32(0)
    for j in range(_NCH):
        c_w = _compact(j, c_w)

    n_chunks = (c_w + 15) >> 4
    total_units = n_chunks * 4

    bufs = ((idx0, ps0, pt0, sps0, spt0), (idx1, ps1, pt1, sps1, spt1))

    def _build_idx(u, idx_ref):
        g = u // 4
        kq = u % 4
        tv = tok_v[g]
        n = lax.shift_right_logical(tv, 6)
        b = tv & 63
        base0 = n * 2048 + (b >> 3) * 256 + (b & 7) + kq * 64
        for j in range(8):
            idx_ref[pl.ds(j * 16, 16)] = base0 + j * 8

    def _fire(u, slot):
        idx_ref, ps_r, pt_r, sps, spt = bufs[slot]
        _build_idx(u, idx_ref)
        pltpu.async_copy(ps_hbm.at[idx_ref], ps_r, sps)
        pltpu.async_copy(pt_hbm.at[idx_ref], pt_r, spt)

    def _wait(slot):
        idx_ref, ps_r, pt_r, sps, spt = bufs[slot]
        pltpu.make_async_copy(ps_hbm.at[idx_ref], ps_r, sps).wait()
        pltpu.make_async_copy(pt_hbm.at[idx_ref], pt_r, spt).wait()

    def _compute(u, slot, carry):
        idx_ref, ps_r, pt_r, sps, spt = bufs[slot]
        g = u // 4
        valid = jnp.minimum(c_w - g * 16, 16)

        def _tok(t, carry):
            accs, zmin = carry
            a = list(accs)
            for j in range(8):
                row = j * 16 + t
                for c in range(8):
                    x = ps_r[row, pl.ds(c * 16, 16)]
                    y = pt_r[row, pl.ds(c * 16, 16)]
                    bits = plsc.bitcast(x, jnp.int32)
                    zmin = jnp.minimum(zmin, bits)
                    e = lax.shift_right_logical(bits, 23)
                    m = plsc.bitcast((bits & 0x007FFFFF) | 0x3F800000,
                                     jnp.float32)
                    l2 = ((_P3 * m + _P2) * m + _P1) * m + (
                        _P0 + e.astype(jnp.float32))
                    a[c] = a[c] + y * l2
            return tuple(a), zmin

        return plsc.parallel_loop(0, valid, carry=carry)(_tok)

    @pl.when(total_units > 0)
    def _():
        _fire(jnp.int32(0), 0)

    def _pair(u2, carry):
        for b in (0, 1):
            u = u2 * 2 + b

            def _do(cr, u=u, b=b):
                @pl.when(u + 1 < total_units)
                def _():
                    _fire(u + 1, 1 - b)

                _wait(b)
                return _compute(u, b, cr)

            carry = lax.cond(u < total_units, _do, lambda cr: cr, carry)
        return carry

    n_pairs = (total_units + 1) // 2
    carry0 = (tuple(jnp.zeros((16,), jnp.float32) for _ in range(8)),
              jnp.full((16,), 0x7FFFFFFF, jnp.int32))
    accs, zmin = lax.fori_loop(0, n_pairs, _pair, carry0)

    acc = accs[0]
    for a in accs[1:]:
        acc = acc + a
    acc = acc * _LN2
    acc = jnp.where(zmin == 0, -jnp.inf, acc)
    accv[...] = acc
    pltpu.sync_copy(accv, out_hbm.at[w])


def kernel(ps, pt, bool_masked_pos):
    pst = jnp.transpose(ps, (1, 0, 2))
    ptt = jnp.transpose(pt, (1, 0, 2))

    def table(xt):
        x5 = xt.reshape(_N, 8, 8, 32, 128)
        return jnp.transpose(x5, (0, 1, 3, 2, 4)).reshape(_TROWS, 128)

    ps2d = table(pst)
    pt2d = table(ptt)

    maskT = jnp.transpose(bool_masked_pos, (1, 0))

    maskf = jnp.broadcast_to(
        maskT.astype(jnp.float32)[:, :, None], (_N, _B, 128))
    tc_out = pl.pallas_call(
        _tc_body,
        grid=(_S // _NB,),
        in_specs=[
            pl.BlockSpec((_NB, _B, 128), lambda i: (i, 0, 0)),
            pl.BlockSpec((_NB, _B, _K), lambda i: (i, 0, 0)),
            pl.BlockSpec((_NB, _B, _K), lambda i: (i, 0, 0)),
        ],
        out_specs=[
            pl.BlockSpec(memory_space=pltpu.SMEM),
            pl.BlockSpec(memory_space=pltpu.SMEM),
        ],
        out_shape=[
            jax.ShapeDtypeStruct((1, 1), jnp.float32),
            jax.ShapeDtypeStruct((1, 1), jnp.float32),
        ],
    )(maskf, pst, ptt)

    mask_pad = jnp.zeros((_TOK_PAD,), jnp.int32).at[:_SC_TOK].set(
        maskT[_S:].reshape(_SC_TOK).astype(jnp.int32))

    mesh = plsc.VectorSubcoreMesh(core_axis_name="c", subcore_axis_name="s")
    sc = functools.partial(
        pl.kernel,
        mesh=mesh,
        compiler_params=pltpu.CompilerParams(needs_layout_passes=False),
        out_type=jax.ShapeDtypeStruct((_NW, 16), jnp.float32),
        scratch_types=[
            pltpu.VMEM((_RW,), jnp.int32),
            pltpu.VMEM((_NCH, 16), jnp.int32),
            pltpu.VMEM((128,), jnp.int32),
            pltpu.VMEM((128,), jnp.int32),
            pltpu.VMEM((128, 128), jnp.float32),
            pltpu.VMEM((128, 128), jnp.float32),
            pltpu.VMEM((128, 128), jnp.float32),
            pltpu.VMEM((128, 128), jnp.float32),
            pltpu.VMEM((16,), jnp.float32),
            pltpu.VMEM((16,), jnp.int32),
            pltpu.SemaphoreType.DMA,
            pltpu.SemaphoreType.DMA,
            pltpu.SemaphoreType.DMA,
            pltpu.SemaphoreType.DMA,
        ],
    )(_sc_body)

    sc_partial = sc(mask_pad, ps2d, pt2d)

    tc_s, _ = tc_out
    total = tc_s[0, 0] + jnp.sum(sc_partial)
    cnt = jnp.sum(bool_masked_pos.astype(jnp.float32))
    return -total / cnt


# speedup vs baseline: 1.0020x; 1.0020x over previous
import functools

import jax
import jax.numpy as jnp
from jax import lax
from jax.experimental import pallas as pl
from jax.experimental.pallas import tpu as pltpu
from jax.experimental.pallas import tpu_sc as plsc

_B, _N, _K = 64, 196, 4096
_NB = 4
_S = 140

_NW = 32
_SC_BASE = _S * _B
_SC_TOK = (_N - _S) * _B
_RW = -(-_SC_TOK // _NW)
_RW = -(-_RW // 16) * 16
_TOK_PAD = _NW * _RW
_NCH = _RW // 16
_TROWS = _N * 8 * 32 * 8

_P0 = -2.133809518617725 - 127.0
_P1 = 3.0107182106126715
_P2 = -1.029486182176504
_P3 = 0.15391242162585833
_LN2 = 0.6931471805599453


def _tc_body(mask_ref, ps_ref, pt_ref, sum_ref, cnt_ref):
    i = pl.program_id(0)

    @pl.when(i == 0)
    def _():
        sum_ref[0, 0] = 0.0
        cnt_ref[0, 0] = 0.0

    m = mask_ref[:, :, 0:1] > 0.0
    safe = jnp.where(m, ps_ref[...], 1.0)
    sum_ref[0, 0] += jnp.sum(pt_ref[...] * jnp.log(safe))
    cnt_ref[0, 0] += jnp.sum(mask_ref[:, :, 0])


def _sc_body(mask_hbm, ps_hbm, pt_hbm, out_hbm,
             mask_v, tok_v, idx0, idx1, ps0, ps1, pt0, pt1, accv, csbuf,
             sps0, sps1, spt0, spt1):
    w = lax.axis_index("s") * 2 + lax.axis_index("c")
    base_tok = w * _RW

    pltpu.sync_copy(mask_hbm.at[pl.ds(base_tok, _RW)], mask_v)

    zeros16 = jnp.zeros((16,), jnp.int32)
    for j in range(8):
        idx0[pl.ds(j * 16, 16)] = zeros16
        idx1[pl.ds(j * 16, 16)] = zeros16
    for j in range(_NCH):
        tok_v[j] = zeros16

    iota16 = lax.iota(jnp.int32, 16)

    def _prefix16(v):
        cur = v
        for d in (1, 2, 4, 8):
            csbuf[...] = cur
            sh = plsc.load_gather(csbuf, [jnp.maximum(iota16 - d, 0)])
            cur = cur + jnp.where(iota16 >= d, sh, 0)
        return cur

    def _compact(j, cum):
        v = mask_v[pl.ds(j * 16, 16)]
        ids = _SC_BASE + base_tok + j * 16 + iota16
        cs = _prefix16(v)
        pos = cum + cs - v
        plsc.store_scatter(tok_v, [pos >> 4, pos & 15], ids, mask=v > 0)
        return cum + lax.squeeze(lax.slice(cs, (15,), (16,)), (0,))

    c_w = jnp.int32(0)
    for j in range(_NCH):
        c_w = _compact(j, c_w)

    n_chunks = (c_w + 15) >> 4
    total_units = n_chunks * 4

    bufs = ((idx0, ps0, pt0, sps0, spt0), (idx1, ps1, pt1, sps1, spt1))

    def _build_idx(u, idx_ref):
        g = u // 4
        kq = u % 4
        tv = tok_v[g]
        n = lax.shift_right_logical(tv, 6)
        b = tv & 63
        base0 = n * 2048 + (b >> 3) * 256 + (b & 7) + kq * 64
        for j in range(8):
            idx_ref[pl.ds(j * 16, 16)] = base0 + j * 8

    def _fire(u, slot):
        idx_ref, ps_r, pt_r, sps, spt = bufs[slot]
        _build_idx(u, idx_ref)
        pltpu.async_copy(ps_hbm.at[idx_ref], ps_r, sps)
        pltpu.async_copy(pt_hbm.at[idx_ref], pt_r, spt)

    def _wait(slot):
        idx_ref, ps_r, pt_r, sps, spt = bufs[slot]
        pltpu.make_async_copy(ps_hbm.at[idx_ref], ps_r, sps).wait()
        pltpu.make_async_copy(pt_hbm.at[idx_ref], pt_r, spt).wait()

    def _compute(u, slot, carry):
        idx_ref, ps_r, pt_r, sps, spt = bufs[slot]
        g = u // 4
        valid = jnp.minimum(c_w - g * 16, 16)

        def _tok(t, carry):
            accs, zmins = carry
            a = list(accs)
            z = list(zmins)
            for j in range(8):
                row = j * 16 + t
                for c in range(8):
                    x = ps_r[row, pl.ds(c * 16, 16)]
                    y = pt_r[row, pl.ds(c * 16, 16)]
                    bits = plsc.bitcast(x, jnp.int32)
                    z[c] = jnp.minimum(z[c], bits)
                    e = lax.shift_right_logical(bits, 23)
                    m = plsc.bitcast((bits & 0x007FFFFF) | 0x3F800000,
                                     jnp.float32)
                    l2 = ((_P3 * m + _P2) * m + _P1) * m + (
                        _P0 + e.astype(jnp.float32))
                    a[c] = a[c] + y * l2
            return tuple(a), tuple(z)

        return plsc.parallel_loop(0, valid, carry=carry)(_tok)

    @pl.when(total_units > 0)
    def _():
        _fire(jnp.int32(0), 0)

    def _pair(u2, carry):
        for b in (0, 1):
            u = u2 * 2 + b

            def _do(cr, u=u, b=b):
                @pl.when(u + 1 < total_units)
                def _():
                    _fire(u + 1, 1 - b)

                _wait(b)
                return _compute(u, b, cr)

            carry = lax.cond(u < total_units, _do, lambda cr: cr, carry)
        return carry

    n_pairs = (total_units + 1) // 2
    carry0 = (tuple(jnp.zeros((16,), jnp.float32) for _ in range(8)),
              tuple(jnp.full((16,), 0x7FFFFFFF, jnp.int32) for _ in range(8)))
    accs, zmins = lax.fori_loop(0, n_pairs, _pair, carry0)

    acc = accs[0]
    for a in accs[1:]:
        acc = acc + a
    zmin = zmins[0]
    for zz in zmins[1:]:
        zmin = jnp.minimum(zmin, zz)
    acc = acc * _LN2
    acc = jnp.where(zmin == 0, -jnp.inf, acc)
    accv[...] = acc
    pltpu.sync_copy(accv, out_hbm.at[w])


def kernel(ps, pt, bool_masked_pos):
    pst = jnp.transpose(ps, (1, 0, 2))
    ptt = jnp.transpose(pt, (1, 0, 2))

    def table(xt):
        x5 = xt.reshape(_N, 8, 8, 32, 128)
        return jnp.transpose(x5, (0, 1, 3, 2, 4)).reshape(_TROWS, 128)

    ps2d = table(pst)
    pt2d = table(ptt)

    maskT = jnp.transpose(bool_masked_pos, (1, 0))

    maskf = jnp.broadcast_to(
        maskT.astype(jnp.float32)[:, :, None], (_N, _B, 128))
    tc_out = pl.pallas_call(
        _tc_body,
        grid=(_S // _NB,),
        in_specs=[
            pl.BlockSpec((_NB, _B, 128), lambda i: (i, 0, 0)),
            pl.BlockSpec((_NB, _B, _K), lambda i: (i, 0, 0)),
            pl.BlockSpec((_NB, _B, _K), lambda i: (i, 0, 0)),
        ],
        out_specs=[
            pl.BlockSpec(memory_space=pltpu.SMEM),
            pl.BlockSpec(memory_space=pltpu.SMEM),
        ],
        out_shape=[
            jax.ShapeDtypeStruct((1, 1), jnp.float32),
            jax.ShapeDtypeStruct((1, 1), jnp.float32),
        ],
    )(maskf, pst, ptt)

    mask_pad = jnp.zeros((_TOK_PAD,), jnp.int32).at[:_SC_TOK].set(
        maskT[_S:].reshape(_SC_TOK).astype(jnp.int32))

    mesh = plsc.VectorSubcoreMesh(core_axis_name="c", subcore_axis_name="s")
    sc = functools.partial(
        pl.kernel,
        mesh=mesh,
        compiler_params=pltpu.CompilerParams(needs_layout_passes=False),
        out_type=jax.ShapeDtypeStruct((_NW, 16), jnp.float32),
        scratch_types=[
            pltpu.VMEM((_RW,), jnp.int32),
            pltpu.VMEM((_NCH, 16), jnp.int32),
            pltpu.VMEM((128,), jnp.int32),
            pltpu.VMEM((128,), jnp.int32),
            pltpu.VMEM((128, 128), jnp.float32),
            pltpu.VMEM((128, 128), jnp.float32),
            pltpu.VMEM((128, 128), jnp.float32),
            pltpu.VMEM((128, 128), jnp.float32),
            pltpu.VMEM((16,), jnp.float32),
            pltpu.VMEM((16,), jnp.int32),
            pltpu.SemaphoreType.DMA,
            pltpu.SemaphoreType.DMA,
            pltpu.SemaphoreType.DMA,
            pltpu.SemaphoreType.DMA,
        ],
    )(_sc_body)

    sc_partial = sc(mask_pad, ps2d, pt2d)

    tc_s, _ = tc_out
    total = tc_s[0, 0] + jnp.sum(sc_partial)
    cnt = jnp.sum(bool_masked_pos.astype(jnp.float32))
    return -total / cnt


# speedup vs baseline: 1.1090x; 1.1069x over previous
import functools

import jax
import jax.numpy as jnp
from jax import lax
from jax.experimental import pallas as pl
from jax.experimental.pallas import tpu as pltpu
from jax.experimental.pallas import tpu_sc as plsc

_B, _N, _K = 64, 196, 4096
_NB = 4
_S = 116

_NW = 32
_SC_BASE = _S * _B
_SC_TOK = (_N - _S) * _B
_RW = -(-_SC_TOK // _NW)
_RW = -(-_RW // 16) * 16
_TOK_PAD = _NW * _RW
_NCH = _RW // 16
_TROWS = _N * 8 * 32 * 8

_P0 = -2.133809518617725 - 127.0
_P1 = 3.0107182106126715
_P2 = -1.029486182176504
_P3 = 0.15391242162585833
_LN2 = 0.6931471805599453


def _tc_body(mask_ref, ps_ref, pt_ref, sum_ref, cnt_ref):
    i = pl.program_id(0)

    @pl.when(i == 0)
    def _():
        sum_ref[0, 0] = 0.0
        cnt_ref[0, 0] = 0.0

    m = mask_ref[:, :, 0:1] > 0.0
    safe = jnp.where(m, ps_ref[...], 1.0)
    sum_ref[0, 0] += jnp.sum(pt_ref[...] * jnp.log(safe))
    cnt_ref[0, 0] += jnp.sum(mask_ref[:, :, 0])


def _sc_body(mask_hbm, ps_hbm, pt_hbm, out_hbm,
             mask_v, tok_v, idx0, idx1, ps0, ps1, pt0, pt1, accv, csbuf,
             sps0, sps1, spt0, spt1):
    w = lax.axis_index("s") * 2 + lax.axis_index("c")
    base_tok = w * _RW

    pltpu.sync_copy(mask_hbm.at[pl.ds(base_tok, _RW)], mask_v)

    zeros16 = jnp.zeros((16,), jnp.int32)
    for j in range(8):
        idx0[pl.ds(j * 16, 16)] = zeros16
        idx1[pl.ds(j * 16, 16)] = zeros16
    for j in range(_NCH):
        tok_v[j] = zeros16

    iota16 = lax.iota(jnp.int32, 16)

    def _prefix16(v):
        cur = v
        for d in (1, 2, 4, 8):
            csbuf[...] = cur
            sh = plsc.load_gather(csbuf, [jnp.maximum(iota16 - d, 0)])
            cur = cur + jnp.where(iota16 >= d, sh, 0)
        return cur

    def _compact(j, cum):
        v = mask_v[pl.ds(j * 16, 16)]
        ids = _SC_BASE + base_tok + j * 16 + iota16
        cs = _prefix16(v)
        pos = cum + cs - v
        plsc.store_scatter(tok_v, [pos >> 4, pos & 15], ids, mask=v > 0)
        return cum + lax.squeeze(lax.slice(cs, (15,), (16,)), (0,))

    c_w = jnp.int32(0)
    for j in range(_NCH):
        c_w = _compact(j, c_w)

    n_chunks = (c_w + 15) >> 4
    total_units = n_chunks * 4

    bufs = ((idx0, ps0, pt0, sps0, spt0), (idx1, ps1, pt1, sps1, spt1))

    def _build_idx(u, idx_ref):
        g = u // 4
        kq = u % 4
        tv = tok_v[g]
        n = lax.shift_right_logical(tv, 6)
        b = tv & 63
        base0 = n * 2048 + (b >> 3) * 256 + (b & 7) + kq * 64
        for j in range(8):
            idx_ref[pl.ds(j * 16, 16)] = base0 + j * 8

    def _fire(u, slot):
        idx_ref, ps_r, pt_r, sps, spt = bufs[slot]
        _build_idx(u, idx_ref)
        pltpu.async_copy(ps_hbm.at[idx_ref], ps_r, sps)
        pltpu.async_copy(pt_hbm.at[idx_ref], pt_r, spt)

    def _wait(slot):
        idx_ref, ps_r, pt_r, sps, spt = bufs[slot]
        pltpu.make_async_copy(ps_hbm.at[idx_ref], ps_r, sps).wait()
        pltpu.make_async_copy(pt_hbm.at[idx_ref], pt_r, spt).wait()

    def _compute(u, slot, carry):
        idx_ref, ps_r, pt_r, sps, spt = bufs[slot]
        g = u // 4
        valid = jnp.minimum(c_w - g * 16, 16)

        def _tok(t, carry):
            accs, zmins = carry
            a = list(accs)
            z = list(zmins)
            for j in range(8):
                row = j * 16 + t
                for c in range(8):
                    x = ps_r[row, pl.ds(c * 16, 16)]
                    y = pt_r[row, pl.ds(c * 16, 16)]
                    bits = plsc.bitcast(x, jnp.int32)
                    z[c] = jnp.minimum(z[c], bits)
                    e = lax.shift_right_logical(bits, 23)
                    m = plsc.bitcast((bits & 0x007FFFFF) | 0x3F800000,
                                     jnp.float32)
                    l2 = ((_P3 * m + _P2) * m + _P1) * m + (
                        _P0 + e.astype(jnp.float32))
                    a[c] = a[c] + y * l2
            return tuple(a), tuple(z)

        return plsc.parallel_loop(0, valid, carry=carry)(_tok)

    @pl.when(total_units > 0)
    def _():
        _fire(jnp.int32(0), 0)

    def _pair(u2, carry):
        for b in (0, 1):
            u = u2 * 2 + b

            def _do(cr, u=u, b=b):
                @pl.when(u + 1 < total_units)
                def _():
                    _fire(u + 1, 1 - b)

                _wait(b)
                return _compute(u, b, cr)

            carry = lax.cond(u < total_units, _do, lambda cr: cr, carry)
        return carry

    n_pairs = (total_units + 1) // 2
    carry0 = (tuple(jnp.zeros((16,), jnp.float32) for _ in range(8)),
              tuple(jnp.full((16,), 0x7FFFFFFF, jnp.int32) for _ in range(8)))
    accs, zmins = lax.fori_loop(0, n_pairs, _pair, carry0)

    acc = accs[0]
    for a in accs[1:]:
        acc = acc + a
    zmin = zmins[0]
    for zz in zmins[1:]:
        zmin = jnp.minimum(zmin, zz)
    acc = acc * _LN2
    acc = jnp.where(zmin == 0, -jnp.inf, acc)
    accv[...] = acc
    pltpu.sync_copy(accv, out_hbm.at[w])


def kernel(ps, pt, bool_masked_pos):
    pst = jnp.transpose(ps, (1, 0, 2))
    ptt = jnp.transpose(pt, (1, 0, 2))

    def table(xt):
        x5 = xt.reshape(_N, 8, 8, 32, 128)
        return jnp.transpose(x5, (0, 1, 3, 2, 4)).reshape(_TROWS, 128)

    ps2d = table(pst)
    pt2d = table(ptt)

    maskT = jnp.transpose(bool_masked_pos, (1, 0))

    maskf = jnp.broadcast_to(
        maskT.astype(jnp.float32)[:, :, None], (_N, _B, 128))
    tc_out = pl.pallas_call(
        _tc_body,
        grid=(_S // _NB,),
        in_specs=[
            pl.BlockSpec((_NB, _B, 128), lambda i: (i, 0, 0)),
            pl.BlockSpec((_NB, _B, _K), lambda i: (i, 0, 0)),
            pl.BlockSpec((_NB, _B, _K), lambda i: (i, 0, 0)),
        ],
        out_specs=[
            pl.BlockSpec(memory_space=pltpu.SMEM),
            pl.BlockSpec(memory_space=pltpu.SMEM),
        ],
        out_shape=[
            jax.ShapeDtypeStruct((1, 1), jnp.float32),
            jax.ShapeDtypeStruct((1, 1), jnp.float32),
        ],
    )(maskf, pst, ptt)

    mask_pad = jnp.zeros((_TOK_PAD,), jnp.int32).at[:_SC_TOK].set(
        maskT[_S:].reshape(_SC_TOK).astype(jnp.int32))

    mesh = plsc.VectorSubcoreMesh(core_axis_name="c", subcore_axis_name="s")
    sc = functools.partial(
        pl.kernel,
        mesh=mesh,
        compiler_params=pltpu.CompilerParams(needs_layout_passes=False),
        out_type=jax.ShapeDtypeStruct((_NW, 16), jnp.float32),
        scratch_types=[
            pltpu.VMEM((_RW,), jnp.int32),
            pltpu.VMEM((_NCH, 16), jnp.int32),
            pltpu.VMEM((128,), jnp.int32),
            pltpu.VMEM((128,), jnp.int32),
            pltpu.VMEM((128, 128), jnp.float32),
            pltpu.VMEM((128, 128), jnp.float32),
            pltpu.VMEM((128, 128), jnp.float32),
            pltpu.VMEM((128, 128), jnp.float32),
            pltpu.VMEM((16,), jnp.float32),
            pltpu.VMEM((16,), jnp.int32),
            pltpu.SemaphoreType.DMA,
            pltpu.SemaphoreType.DMA,
            pltpu.SemaphoreType.DMA,
            pltpu.SemaphoreType.DMA,
        ],
    )(_sc_body)

    sc_partial = sc(mask_pad, ps2d, pt2d)

    tc_s, _ = tc_out
    total = tc_s[0, 0] + jnp.sum(sc_partial)
    cnt = jnp.sum(bool_masked_pos.astype(jnp.float32))
    return -total / cnt


# speedup vs baseline: 1.1302x; 1.0191x over previous
import functools

import jax
import jax.numpy as jnp
from jax import lax
from jax.experimental import pallas as pl
from jax.experimental.pallas import tpu as pltpu
from jax.experimental.pallas import tpu_sc as plsc

_B, _N, _K = 64, 196, 4096
_NB = 4
_S = 108

_NW = 32
_SC_BASE = _S * _B
_SC_TOK = (_N - _S) * _B
_RW = -(-_SC_TOK // _NW)
_RW = -(-_RW // 16) * 16
_TOK_PAD = _NW * _RW
_NCH = _RW // 16
_TROWS = _N * 8 * 32 * 8

_P0 = -2.133809518617725 - 127.0
_P1 = 3.0107182106126715
_P2 = -1.029486182176504
_P3 = 0.15391242162585833
_LN2 = 0.6931471805599453


def _tc_body(mask_ref, ps_ref, pt_ref, sum_ref, cnt_ref):
    i = pl.program_id(0)

    @pl.when(i == 0)
    def _():
        sum_ref[0, 0] = 0.0
        cnt_ref[0, 0] = 0.0

    m = mask_ref[:, :, 0:1] > 0.0
    safe = jnp.where(m, ps_ref[...], 1.0)
    sum_ref[0, 0] += jnp.sum(pt_ref[...] * jnp.log(safe))
    cnt_ref[0, 0] += jnp.sum(mask_ref[:, :, 0])


def _sc_body(mask_hbm, ps_hbm, pt_hbm, out_hbm,
             mask_v, tok_v, idx0, idx1, ps0, ps1, pt0, pt1, accv, csbuf,
             sps0, sps1, spt0, spt1):
    w = lax.axis_index("s") * 2 + lax.axis_index("c")
    base_tok = w * _RW

    pltpu.sync_copy(mask_hbm.at[pl.ds(base_tok, _RW)], mask_v)

    zeros16 = jnp.zeros((16,), jnp.int32)
    for j in range(8):
        idx0[pl.ds(j * 16, 16)] = zeros16
        idx1[pl.ds(j * 16, 16)] = zeros16
    for j in range(_NCH):
        tok_v[j] = zeros16

    iota16 = lax.iota(jnp.int32, 16)

    def _prefix16(v):
        cur = v
        for d in (1, 2, 4, 8):
            csbuf[...] = cur
            sh = plsc.load_gather(csbuf, [jnp.maximum(iota16 - d, 0)])
            cur = cur + jnp.where(iota16 >= d, sh, 0)
        return cur

    def _compact(j, cum):
        v = mask_v[pl.ds(j * 16, 16)]
        ids = _SC_BASE + base_tok + j * 16 + iota16
        cs = _prefix16(v)
        pos = cum + cs - v
        plsc.store_scatter(tok_v, [pos >> 4, pos & 15], ids, mask=v > 0)
        return cum + lax.squeeze(lax.slice(cs, (15,), (16,)), (0,))

    c_w = jnp.int32(0)
    for j in range(_NCH):
        c_w = _compact(j, c_w)

    n_chunks = (c_w + 15) >> 4
    total_units = n_chunks * 4

    bufs = ((idx0, ps0, pt0, sps0, spt0), (idx1, ps1, pt1, sps1, spt1))

    def _build_idx(u, idx_ref):
        g = u // 4
        kq = u % 4
        tv = tok_v[g]
        n = lax.shift_right_logical(tv, 6)
        b = tv & 63
        base0 = n * 2048 + (b >> 3) * 256 + (b & 7) + kq * 64
        for j in range(8):
            idx_ref[pl.ds(j * 16, 16)] = base0 + j * 8

    def _fire(u, slot):
        idx_ref, ps_r, pt_r, sps, spt = bufs[slot]
        _build_idx(u, idx_ref)
        pltpu.async_copy(ps_hbm.at[idx_ref], ps_r, sps)
        pltpu.async_copy(pt_hbm.at[idx_ref], pt_r, spt)

    def _wait(slot):
        idx_ref, ps_r, pt_r, sps, spt = bufs[slot]
        pltpu.make_async_copy(ps_hbm.at[idx_ref], ps_r, sps).wait()
        pltpu.make_async_copy(pt_hbm.at[idx_ref], pt_r, spt).wait()

    def _compute(u, slot, carry):
        idx_ref, ps_r, pt_r, sps, spt = bufs[slot]
        g = u // 4
        valid = jnp.minimum(c_w - g * 16, 16)

        def _tok(t, carry):
            accs, zmins = carry
            a = list(accs)
            z = list(zmins)
            for j in range(8):
                row = j * 16 + t
                for c in range(8):
                    x = ps_r[row, pl.ds(c * 16, 16)]
                    y = pt_r[row, pl.ds(c * 16, 16)]
                    bits = plsc.bitcast(x, jnp.int32)
                    z[c] = jnp.minimum(z[c], bits)
                    e = lax.shift_right_logical(bits, 23)
                    m = plsc.bitcast((bits & 0x007FFFFF) | 0x3F800000,
                                     jnp.float32)
                    l2 = ((_P3 * m + _P2) * m + _P1) * m + (
                        _P0 + e.astype(jnp.float32))
                    a[c] = a[c] + y * l2
            return tuple(a), tuple(z)

        return plsc.parallel_loop(0, valid, carry=carry)(_tok)

    @pl.when(total_units > 0)
    def _():
        _fire(jnp.int32(0), 0)

    def _pair(u2, carry):
        for b in (0, 1):
            u = u2 * 2 + b

            def _do(cr, u=u, b=b):
                @pl.when(u + 1 < total_units)
                def _():
                    _fire(u + 1, 1 - b)

                _wait(b)
                return _compute(u, b, cr)

            carry = lax.cond(u < total_units, _do, lambda cr: cr, carry)
        return carry

    n_pairs = (total_units + 1) // 2
    carry0 = (tuple(jnp.zeros((16,), jnp.float32) for _ in range(8)),
              tuple(jnp.full((16,), 0x7FFFFFFF, jnp.int32) for _ in range(8)))
    accs, zmins = lax.fori_loop(0, n_pairs, _pair, carry0)

    acc = accs[0]
    for a in accs[1:]:
        acc = acc + a
    zmin = zmins[0]
    for zz in zmins[1:]:
        zmin = jnp.minimum(zmin, zz)
    acc = acc * _LN2
    acc = jnp.where(zmin == 0, -jnp.inf, acc)
    accv[...] = acc
    pltpu.sync_copy(accv, out_hbm.at[w])


def kernel(ps, pt, bool_masked_pos):
    pst = jnp.transpose(ps, (1, 0, 2))
    ptt = jnp.transpose(pt, (1, 0, 2))

    def table(xt):
        x5 = xt.reshape(_N, 8, 8, 32, 128)
        return jnp.transpose(x5, (0, 1, 3, 2, 4)).reshape(_TROWS, 128)

    ps2d = table(pst)
    pt2d = table(ptt)

    maskT = jnp.transpose(bool_masked_pos, (1, 0))

    maskf = jnp.broadcast_to(
        maskT.astype(jnp.float32)[:, :, None], (_N, _B, 128))
    tc_out = pl.pallas_call(
        _tc_body,
        grid=(_S // _NB,),
        in_specs=[
            pl.BlockSpec((_NB, _B, 128), lambda i: (i, 0, 0)),
            pl.BlockSpec((_NB, _B, _K), lambda i: (i, 0, 0)),
            pl.BlockSpec((_NB, _B, _K), lambda i: (i, 0, 0)),
        ],
        out_specs=[
            pl.BlockSpec(memory_space=pltpu.SMEM),
            pl.BlockSpec(memory_space=pltpu.SMEM),
        ],
        out_shape=[
            jax.ShapeDtypeStruct((1, 1), jnp.float32),
            jax.ShapeDtypeStruct((1, 1), jnp.float32),
        ],
    )(maskf, pst, ptt)

    mask_pad = jnp.zeros((_TOK_PAD,), jnp.int32).at[:_SC_TOK].set(
        maskT[_S:].reshape(_SC_TOK).astype(jnp.int32))

    mesh = plsc.VectorSubcoreMesh(core_axis_name="c", subcore_axis_name="s")
    sc = functools.partial(
        pl.kernel,
        mesh=mesh,
        compiler_params=pltpu.CompilerParams(needs_layout_passes=False),
        out_type=jax.ShapeDtypeStruct((_NW, 16), jnp.float32),
        scratch_types=[
            pltpu.VMEM((_RW,), jnp.int32),
            pltpu.VMEM((_NCH, 16), jnp.int32),
            pltpu.VMEM((128,), jnp.int32),
            pltpu.VMEM((128,), jnp.int32),
            pltpu.VMEM((128, 128), jnp.float32),
            pltpu.VMEM((128, 128), jnp.float32),
            pltpu.VMEM((128, 128), jnp.float32),
            pltpu.VMEM((128, 128), jnp.float32),
            pltpu.VMEM((16,), jnp.float32),
            pltpu.VMEM((16,), jnp.int32),
            pltpu.SemaphoreType.DMA,
            pltpu.SemaphoreType.DMA,
            pltpu.SemaphoreType.DMA,
            pltpu.SemaphoreType.DMA,
        ],
    )(_sc_body)

    sc_partial = sc(mask_pad, ps2d, pt2d)

    tc_s, _ = tc_out
    total = tc_s[0, 0] + jnp.sum(sc_partial)
    cnt = jnp.sum(bool_masked_pos.astype(jnp.float32))
    return -total / cnt
